# baseline (speedup 1.0000x reference)
import functools

import jax
import jax.numpy as jnp
from jax import lax
from jax.experimental import pallas as pl
from jax.experimental.pallas import tpu as pltpu
from jax.experimental.pallas import tpu_sc as plsc

_SC_CORES = 2
_SC_SUBCORES = 16
_SC_WORKERS = _SC_CORES * _SC_SUBCORES
_LANES = 16
_SC_BLOCK_ROWS = 32


def _sc_partial_sums(pred2, gt2):
    rows, C = pred2.shape
    groups = C // _LANES
    mesh = plsc.VectorSubcoreMesh(core_axis_name="c", subcore_axis_name="s")

    @functools.partial(
        pl.kernel,
        mesh=mesh,
        out_type=jax.ShapeDtypeStruct((_SC_WORKERS, _LANES), jnp.float32),
        scratch_types=[pltpu.VMEM((_LANES,), jnp.float32)],
    )
    def k(pred_hbm, gt_hbm, out_hbm, acc_ref):
        wid = lax.axis_index("c") * _SC_SUBCORES + lax.axis_index("s")
        acc_ref[...] = jnp.zeros((_LANES,), jnp.float32)

        def body(pred_v, gt_v):
            @pl.loop(0, _SC_BLOCK_ROWS)
            def _(r):
                s = jnp.zeros((_LANES,), jnp.float32)
                g0 = None
                for c in range(groups):
                    sl = pl.ds(c * _LANES, _LANES)
                    g = gt_v[r, sl]
                    if c == 0:
                        g0 = g[0]
                    d = pred_v[r, sl] - g
                    s = s + d * d
                m = jnp.where(g0 != -1.0, 1.0, 0.0)
                acc_ref[...] = acc_ref[...] + s * m

        pltpu.emit_pipeline(
            body,
            grid=(rows // _SC_BLOCK_ROWS,),
            in_specs=[
                pl.BlockSpec((_SC_BLOCK_ROWS, C), lambda i: (i, 0)),
                pl.BlockSpec((_SC_BLOCK_ROWS, C), lambda i: (i, 0)),
            ],
            out_specs=[],
            core_axis_name=("c", "s"),
            dimension_semantics=(pltpu.PARALLEL,),
        )(pred_hbm, gt_hbm)

        pltpu.sync_copy(acc_ref, out_hbm.at[wid])

    return k(pred2, gt2)


def kernel(pred, gt):
    B, N, C = pred.shape
    rows = B * N
    pred2 = pred.reshape(rows, C)
    gt2 = gt.reshape(rows, C)
    partials = _sc_partial_sums(pred2, gt2)
    return jnp.sum(partials)
